# baseline (speedup 1.0000x reference)
import functools

import jax
import jax.numpy as jnp
from jax import lax
from jax.experimental import pallas as pl
from jax.experimental.pallas import tpu as pltpu
from jax.experimental.pallas import tpu_sc as plsc

NC = 2
NS = 16
NW = NC * NS

CH = 128
L = 16
KC = 2048


def _repack_table(emb_t):
    d, v = emb_t.shape
    nb = (v + KC - 1) // KC

    def body(x_ref, o_ref):
        o_ref[:, 0:d] = jnp.swapaxes(x_ref[...], 0, 1)

    return pl.pallas_call(
        body,
        grid=(nb,),
        in_specs=[pl.BlockSpec((d, KC), lambda j: (0, j))],
        out_specs=pl.BlockSpec((KC, 128), lambda j: (j, 0)),
        out_shape=jax.ShapeDtypeStruct((v, 128), jnp.float32),
    )(emb_t)


def _emb_lookup(table, idx_t, *, b, h, d):
    n = b * h
    T = n // (NW * CH)
    jb_n = b // CH

    mesh = plsc.VectorSubcoreMesh(core_axis_name="c", subcore_axis_name="s")

    @functools.partial(
        pl.kernel,
        out_type=jax.ShapeDtypeStruct((h * d, b), jnp.float32),
        mesh=mesh,
        scratch_types=[
            pltpu.VMEM((CH,), jnp.int32),
            pltpu.VMEM((CH,), jnp.int32),
            pltpu.VMEM((CH, 128), jnp.float32),
            pltpu.VMEM((CH, 128), jnp.float32),
            pltpu.VMEM((d, CH), jnp.float32),
            pltpu.SemaphoreType.DMA,
        ],
        compiler_params=pltpu.CompilerParams(needs_layout_passes=False),
    )
    def body(table_hbm, idx_hbm, out_hbm, iv0, iv1, g0, g1, o_v, gsem):
        wid = lax.axis_index("s") * NC + lax.axis_index("c")
        c0 = wid * T

        def load_fire(t, iv, gbuf):
            cid = c0 + t
            hh = lax.div(cid, jb_n)
            jb = lax.rem(cid, jb_n)
            pltpu.sync_copy(idx_hbm.at[hh, pl.ds(jb * CH, CH)], iv)
            pltpu.async_copy(table_hbm.at[iv], gbuf, gsem)

        def drain(iv, gbuf):
            pltpu.make_async_copy(table_hbm.at[iv], gbuf, gsem).wait()

        def transpose_put(t, gbuf):
            for c in range(d):
                cols = jnp.full((L,), c, jnp.int32)
                for g in range(CH // L):
                    rows = lax.iota(jnp.int32, L) + (g * L)
                    vals = plsc.load_gather(gbuf, [rows, cols])
                    o_v[c, pl.ds(g * L, L)] = vals
            cid = c0 + t
            hh = lax.div(cid, jb_n)
            jb = lax.rem(cid, jb_n)
            pltpu.sync_copy(
                o_v, out_hbm.at[pl.ds(hh * d, d), pl.ds(jb * CH, CH)]
            )

        load_fire(0, iv0, g0)

        @pl.loop(0, T // 2)
        def _(tt):
            t0 = tt * 2
            load_fire(t0 + 1, iv1, g1)
            drain(iv0, g0)
            transpose_put(t0, g0)

            @pl.when(t0 + 2 < T)
            def _():
                load_fire(t0 + 2, iv0, g0)

            drain(iv1, g1)
            transpose_put(t0 + 1, g1)

    return body(table, idx_t)


def kernel(emb_mat, input):
    v, d = emb_mat.shape
    b, h = input.shape
    table = _repack_table(emb_mat.T)
    idx_t = input.T.astype(jnp.int32)
    out2d = _emb_lookup(table, idx_t, b=b, h=h, d=d)
    return out2d.reshape(h, d, b).transpose(2, 0, 1)


# speedup vs baseline: 1.6102x; 1.6102x over previous
import functools

import jax
import jax.numpy as jnp
from jax import lax
from jax.experimental import pallas as pl
from jax.experimental.pallas import tpu as pltpu
from jax.experimental.pallas import tpu_sc as plsc

NC = 2
NS = 16
NW = NC * NS

CH = 128
L = 16
KC = 2048


def _repack_table(emb_t, vh):
    d, v = emb_t.shape
    nh = vh // KC
    nb_last = (v + KC - 1) // KC - 1

    def body(xl_ref, xh_ref, o_ref):
        o_ref[:, 0:d] = jnp.swapaxes(xl_ref[...], 0, 1)
        o_ref[:, d:2 * d] = jnp.swapaxes(xh_ref[...], 0, 1)

    return pl.pallas_call(
        body,
        grid=(nh,),
        in_specs=[
            pl.BlockSpec((d, KC), lambda j: (0, j)),
            pl.BlockSpec((d, KC), lambda j: (0, jnp.minimum(j + nh, nb_last))),
        ],
        out_specs=pl.BlockSpec((KC, 2 * d), lambda j: (j, 0)),
        out_shape=jax.ShapeDtypeStruct((vh, 2 * d), jnp.float32),
    )(emb_t, emb_t)


def _prep_idx(idx2, sh):
    def body(x_ref, m_ref, p_ref):
        x = x_ref[...]
        m_ref[...] = jnp.bitwise_and(x, (1 << sh) - 1)
        p_ref[...] = lax.shift_left(lax.shift_right_logical(x, sh), 6)

    return pl.pallas_call(
        body,
        out_shape=(
            jax.ShapeDtypeStruct(idx2.shape, jnp.int32),
            jax.ShapeDtypeStruct(idx2.shape, jnp.int32),
        ),
    )(idx2)


def _emb_lookup(table, idxm3, idxp3, *, b, h, d):
    n = b * h
    T = n // (NW * CH)
    jb_n = b // CH

    mesh = plsc.VectorSubcoreMesh(core_axis_name="c", subcore_axis_name="s")

    @functools.partial(
        pl.kernel,
        out_type=jax.ShapeDtypeStruct((h * d, b), jnp.float32),
        mesh=mesh,
        scratch_types=[
            pltpu.VMEM((T, CH), jnp.int32),
            pltpu.VMEM((T, CH), jnp.int32),
            pltpu.VMEM((CH, 2 * d), jnp.float32),
            pltpu.VMEM((CH, 2 * d), jnp.float32),
            pltpu.VMEM((d, CH), jnp.float32),
            pltpu.SemaphoreType.DMA,
        ],
        compiler_params=pltpu.CompilerParams(needs_layout_passes=False),
    )
    def body(table_hbm, idxm_hbm, idxp_hbm, out_hbm, ivh, ivp, g0, g1,
             o_v, gsem):
        wid = lax.axis_index("s") * NC + lax.axis_index("c")
        c0 = wid * T

        pltpu.sync_copy(idxm_hbm.at[wid], ivh)
        pltpu.sync_copy(idxp_hbm.at[wid], ivp)

        def fire(t, gbuf):
            pltpu.async_copy(table_hbm.at[ivh.at[t]], gbuf, gsem)

        def drain(t, gbuf):
            pltpu.make_async_copy(table_hbm.at[ivh.at[t]], gbuf, gsem).wait()

        def transpose_put(t, gbuf):
            for g in range(CH // L):
                rows = lax.iota(jnp.int32, L) + (g * L)
                sl = pl.ds(g * L, L)
                par = ivp[t, sl]
                for cb in range(0, d, 8):
                    vals = [
                        plsc.load_gather(gbuf, [rows, par + c])
                        for c in range(cb, cb + 8)
                    ]
                    for k in range(8):
                        o_v[cb + k, sl] = vals[k]
            cid = c0 + t
            hh = lax.div(cid, jb_n)
            jb = lax.rem(cid, jb_n)
            pltpu.sync_copy(
                o_v, out_hbm.at[pl.ds(hh * d, d), pl.ds(jb * CH, CH)]
            )

        fire(0, g0)

        @pl.loop(0, T // 2)
        def _(tt):
            t0 = tt * 2
            fire(t0 + 1, g1)
            drain(t0, g0)
            transpose_put(t0, g0)

            @pl.when(t0 + 2 < T)
            def _():
                fire(t0 + 2, g0)

            drain(t0 + 1, g1)
            transpose_put(t0 + 1, g1)

    return body(table, idxm3, idxp3)


def kernel(emb_mat, input):
    v, d = emb_mat.shape
    b, h = input.shape
    n = b * h
    T = n // (NW * CH)
    sh = max((v - 1).bit_length() - 1, 0)
    table = _repack_table(emb_mat.T, 1 << sh)
    idx2 = input.T.reshape(n // CH, CH).astype(jnp.int32)
    idxm, idxp = _prep_idx(idx2, sh)
    idxm3 = idxm.reshape(NW, T, CH)
    idxp3 = idxp.reshape(NW, T, CH)
    out2d = _emb_lookup(table, idxm3, idxp3, b=b, h=h, d=d)
    return out2d.reshape(h, d, b).transpose(2, 0, 1)


# speedup vs baseline: 2.0177x; 1.2531x over previous
import functools

import jax
import jax.numpy as jnp
from jax import lax
from jax.experimental import pallas as pl
from jax.experimental.pallas import tpu as pltpu
from jax.experimental.pallas import tpu_sc as plsc

NC = 2
NS = 16
NW = NC * NS

CH = 128
L = 16
KC = 2048


def _repack_table(emb_t, vh):
    d, v = emb_t.shape
    nh = vh // KC
    nb_last = (v + KC - 1) // KC - 1

    def body(xl_ref, xh_ref, o_ref):
        o_ref[:, 0:d] = jnp.swapaxes(xl_ref[...], 0, 1)
        o_ref[:, d:2 * d] = jnp.swapaxes(xh_ref[...], 0, 1)

    return pl.pallas_call(
        body,
        grid=(nh,),
        in_specs=[
            pl.BlockSpec((d, KC), lambda j: (0, j)),
            pl.BlockSpec((d, KC), lambda j: (0, jnp.minimum(j + nh, nb_last))),
        ],
        out_specs=pl.BlockSpec((KC, 2 * d), lambda j: (j, 0)),
        out_shape=jax.ShapeDtypeStruct((vh, 2 * d), jnp.float32),
    )(emb_t, emb_t)


def _prep_idx(idx2, sh):
    def body(x_ref, m_ref, p_ref):
        x = x_ref[...]
        m_ref[...] = jnp.bitwise_and(x, (1 << sh) - 1)
        p_ref[...] = lax.shift_left(lax.shift_right_logical(x, sh), 6)

    return pl.pallas_call(
        body,
        out_shape=(
            jax.ShapeDtypeStruct(idx2.shape, jnp.int32),
            jax.ShapeDtypeStruct(idx2.shape, jnp.int32),
        ),
    )(idx2)


def _emb_lookup(table, idxm3, idxp3, *, b, h, d):
    n = b * h
    T = n // (NW * CH)
    jb_n = b // CH

    mesh = plsc.VectorSubcoreMesh(core_axis_name="c", subcore_axis_name="s")

    @functools.partial(
        pl.kernel,
        out_type=jax.ShapeDtypeStruct((h * d, b), jnp.float32),
        mesh=mesh,
        scratch_types=[
            pltpu.VMEM((T, CH), jnp.int32),
            pltpu.VMEM((T, CH), jnp.int32),
            pltpu.VMEM((CH, 2 * d), jnp.float32),
            pltpu.VMEM((CH, 2 * d), jnp.float32),
            pltpu.VMEM((d, CH), jnp.float32),
            pltpu.SemaphoreType.DMA,
        ],
        compiler_params=pltpu.CompilerParams(needs_layout_passes=False),
    )
    def body(table_hbm, idxm_hbm, idxp_hbm, out_hbm, ivh, ivp, g0, g1,
             o_v, gsem):
        wid = lax.axis_index("s") * NC + lax.axis_index("c")
        c0 = wid * T

        pltpu.sync_copy(idxm_hbm.at[wid], ivh)
        pltpu.sync_copy(idxp_hbm.at[wid], ivp)

        def fire(t, gbuf):
            pltpu.async_copy(table_hbm.at[ivh.at[t]], gbuf, gsem)

        def drain(t, gbuf):
            pltpu.make_async_copy(table_hbm.at[ivh.at[t]], gbuf, gsem).wait()

        kvec = lax.iota(jnp.int32, L)
        rowvs = [kvec + (jg * L) for jg in range(CH // L)]

        def transpose_put(t, gbuf):
            pars = [ivp[t, pl.ds(jg * L, L)] for jg in range(CH // L)]

            @pl.loop(0, L)
            def _(s):
                perm = jnp.bitwise_and(kvec + s, L - 1)
                for eg in range(d // L):
                    pe = perm + (eg * L)
                    for jg in range(CH // L):
                        vals = plsc.load_gather(
                            gbuf, [rowvs[jg], pe + pars[jg]]
                        )
                        plsc.store_scatter(o_v, [pe, rowvs[jg]], vals)

            cid = c0 + t
            hh = lax.div(cid, jb_n)
            jb = lax.rem(cid, jb_n)
            pltpu.sync_copy(
                o_v, out_hbm.at[pl.ds(hh * d, d), pl.ds(jb * CH, CH)]
            )

        fire(0, g0)

        @pl.loop(0, T // 2)
        def _(tt):
            t0 = tt * 2
            fire(t0 + 1, g1)
            drain(t0, g0)
            transpose_put(t0, g0)

            @pl.when(t0 + 2 < T)
            def _():
                fire(t0 + 2, g0)

            drain(t0 + 1, g1)
            transpose_put(t0 + 1, g1)

    return body(table, idxm3, idxp3)


def kernel(emb_mat, input):
    v, d = emb_mat.shape
    b, h = input.shape
    n = b * h
    T = n // (NW * CH)
    sh = max((v - 1).bit_length() - 1, 0)
    table = _repack_table(emb_mat.T, 1 << sh)
    idx2 = input.T.reshape(n // CH, CH).astype(jnp.int32)
    idxm, idxp = _prep_idx(idx2, sh)
    idxm3 = idxm.reshape(NW, T, CH)
    idxp3 = idxp.reshape(NW, T, CH)
    out2d = _emb_lookup(table, idxm3, idxp3, b=b, h=h, d=d)
    return out2d.reshape(h, d, b).transpose(2, 0, 1)
